# baseline (speedup 1.0000x reference)
import jax
import jax.numpy as jnp
from jax import lax
from jax.experimental import pallas as pl
from jax.experimental.pallas import tpu as pltpu
from jax.experimental.pallas import tpu_sc as plsc

N = 10000
E = 320000
D = 128
ALPHA = 0.2

NC = 2
NS = 16
NW = NC * NS
TPW = E // NW
BLK = 48
NBLK = 216
NGRP = NBLK // 8
TPAD = NBLK * BLK
NPAD = 79 * 128


def _phase_a_body(x_ref, ww_ref, wb_ref, aw_ref, ab_ref, wh_ref,
                  p_ref, q_ref, mst_ref):
    x = x_ref[...]
    wh = lax.dot_general(x, ww_ref[...], (((1,), (1,)), ((), ())),
                         preferred_element_type=jnp.float32)
    wh = wh + wb_ref[...][None, :]
    wh_ref[...] = wh
    a2 = aw_ref[...].reshape(2, D)
    pq = lax.dot_general(wh, a2, (((1,), (1,)), ((), ())),
                         preferred_element_type=jnp.float32)
    p = pq[:, 0] + ab_ref[0]
    q = pq[:, 1]
    pad = jnp.zeros((NPAD - N,), jnp.float32)
    p_ref[...] = jnp.concatenate([p, pad])
    q_ref[...] = jnp.concatenate([q, pad])
    m = jnp.max(p) + jnp.max(q)
    m = jnp.where(m < 0, ALPHA * m, m)
    mst_ref[...] = jnp.full((16,), m, jnp.float32)


def _phase_a(x, W_w, W_b, a_w, a_b):
    return pl.pallas_call(
        _phase_a_body,
        out_shape=(
            jax.ShapeDtypeStruct((N, D), jnp.float32),
            jax.ShapeDtypeStruct((NPAD,), jnp.float32),
            jax.ShapeDtypeStruct((NPAD,), jnp.float32),
            jax.ShapeDtypeStruct((16,), jnp.float32),
        ),
    )(x, W_w, W_b, a_w, a_b)


def _sc_body(wh_hbm, p_hbm, q_hbm, src_hbm, dst_hbm, turn_hbm, tv_hbm,
             mst_hbm, num_out, den_out,
             num_sh, idx_s, idx_d, idx_t, p_v, q_v, den_v,
             rows0, rows1, mst_v, tv_v,
             semr0, semr1, semx):
    cid = lax.axis_index("c")
    sid = lax.axis_index("s")
    wid = cid * NS + sid

    pltpu.sync_copy(mst_hbm, mst_v)
    pltpu.sync_copy(tv_hbm, tv_v)
    pltpu.sync_copy(p_hbm, p_v)
    pltpu.sync_copy(q_hbm, q_v)

    zeros16f = jnp.zeros((16,), jnp.float32)

    def _zero_den(i, _):
        den_v[pl.ds(i * 16, 16)] = zeros16f
        return 0
    lax.fori_loop(0, NPAD // 16, _zero_den, 0)

    def _zero_rows(i, _):
        r = i // 8
        c = (i % 8) * 16
        rows0[r, pl.ds(c, 16)] = zeros16f
        return 0
    lax.fori_loop(0, BLK * 8, _zero_rows, 0)

    base = sid * (NPAD // NS)

    def _zero_sh(i, _):
        pltpu.sync_copy(rows0, num_sh.at[pl.ds(base + i * BLK, BLK)])
        return 0
    lax.fori_loop(0, 13, _zero_sh, 0)
    pltpu.sync_copy(rows0.at[pl.ds(0, 8)],
                    num_sh.at[pl.ds(base + 13 * BLK, 8)])

    mst = mst_v[...]
    tv = tv_v[...]

    plsc.subcore_barrier()


    def issue_group_load(gi):
        gb = gi % 2
        pltpu.async_copy(src_hbm.at[wid, gi], idx_s.at[gb], semx)
        pltpu.async_copy(dst_hbm.at[wid, gi], idx_d.at[gb], semx)
        pltpu.async_copy(turn_hbm.at[wid, gi], idx_t.at[gb], semx)

    def wait_group_load():
        pltpu.make_async_copy(src_hbm.at[wid, 0], idx_s.at[0], semx).wait()
        pltpu.make_async_copy(dst_hbm.at[wid, 0], idx_d.at[0], semx).wait()
        pltpu.make_async_copy(turn_hbm.at[wid, 0], idx_t.at[0], semx).wait()

    def issue_gather(j, rows, semr):
        gp = (j // 8) % 2
        jj = j % 8
        pltpu.async_copy(wh_hbm.at[idx_s.at[gp, jj]], rows, semr)

    def wait_gather(rows, semr):
        pltpu.make_async_copy(wh_hbm.at[idx_s.at[0, 0]], rows, semr).wait()

    def compute_block(j, rows):
        gp = (j // 8) % 2
        jj = j % 8

        def _sub(g, _c):
            c = g * 16
            s16 = idx_s[gp, jj, pl.ds(c, 16)]
            d16 = idx_d[gp, jj, pl.ds(c, 16)]
            t16 = idx_t[gp, jj, pl.ds(c, 16)]
            pv = plsc.load_gather(p_v, [s16])
            qv = plsc.load_gather(q_v, [d16])
            e = pv + qv
            e = jnp.where(e < 0, ALPHA * e, e)
            w = jnp.exp(e - mst)
            w = jnp.where((t16 == tv) & (d16 < N), w, 0.0)
            plsc.addupdate_scatter(den_v, [d16], w)
            for i in range(16):
                wsc = w[i]
                r = c + i
                for k in range(8):
                    sl = pl.ds(k * 16, 16)
                    rows[r, sl] = rows[r, sl] * wsc
            return 0
        lax.fori_loop(0, BLK // 16, _sub, 0)

    def scatter_block(j, rows):
        gp = (j // 8) % 2
        jj = j % 8
        pltpu.sync_copy(rows, num_sh.at[idx_d.at[gp, jj]], add=True)

    pltpu.sync_copy(src_hbm.at[wid, 0], idx_s.at[0])
    pltpu.sync_copy(dst_hbm.at[wid, 0], idx_d.at[0])
    pltpu.sync_copy(turn_hbm.at[wid, 0], idx_t.at[0])
    issue_group_load(1)
    issue_gather(0, rows0, semr0)

    def _pair(it, _):
        j0 = 2 * it
        j1 = j0 + 1

        @pl.when((it % 4 == 0) & (it > 0) & (it // 4 + 1 <= NGRP - 1))
        def _():
            issue_group_load(it // 4 + 1)

        issue_gather(j1, rows1, semr1)
        wait_gather(rows0, semr0)
        compute_block(j0, rows0)
        scatter_block(j0, rows0)

        @pl.when(j0 + 2 < NBLK)
        def _():
            @pl.when((j0 + 2) % 8 == 0)
            def _():
                wait_group_load()
            issue_gather(j0 + 2, rows0, semr0)

        wait_gather(rows1, semr1)
        compute_block(j1, rows1)
        scatter_block(j1, rows1)
        return 0
    lax.fori_loop(0, NBLK // 2, _pair, 0)

    pltpu.sync_copy(den_v, den_out.at[pl.ds(wid * NPAD, NPAD)])

    plsc.subcore_barrier()

    rows_per_tile = NPAD // NS
    pltpu.sync_copy(num_sh.at[pl.ds(base, rows_per_tile)],
                    num_out.at[cid, pl.ds(base, rows_per_tile)])


def _phase_b(wh, p, q, srcp, dstp, turnp, tv, mst):
    mesh = plsc.VectorSubcoreMesh(core_axis_name="c", subcore_axis_name="s",
                                  num_cores=NC, num_subcores=NS)
    f = pl.kernel(
        _sc_body,
        out_type=(
            jax.ShapeDtypeStruct((NC, NPAD, D), jnp.float32),
            jax.ShapeDtypeStruct((NW * NPAD,), jnp.float32),
        ),
        mesh=mesh,
        scratch_types=[
            pltpu.VMEM_SHARED((NPAD, D), jnp.float32),
            pltpu.VMEM((2, 8, BLK), jnp.int32),
            pltpu.VMEM((2, 8, BLK), jnp.int32),
            pltpu.VMEM((2, 8, BLK), jnp.int32),
            pltpu.VMEM((NPAD,), jnp.float32),
            pltpu.VMEM((NPAD,), jnp.float32),
            pltpu.VMEM((NPAD,), jnp.float32),
            pltpu.VMEM((BLK, D), jnp.float32),
            pltpu.VMEM((BLK, D), jnp.float32),
            pltpu.VMEM((16,), jnp.float32),
            pltpu.VMEM((16,), jnp.int32),
            pltpu.SemaphoreType.DMA,
            pltpu.SemaphoreType.DMA,
            pltpu.SemaphoreType.DMA,
        ],
        compiler_params=pltpu.CompilerParams(needs_layout_passes=False),
    )
    return f(wh, p, q, srcp, dstp, turnp, tv, mst)


def _phase_c_body(num_ref, den_ref, out_ref):
    num = num_ref[0] + num_ref[1]
    den = jnp.sum(den_ref[...], axis=0)
    out_ref[...] = num / (den[:, None] + 1e-9)


def _phase_c(num, den):
    return pl.pallas_call(
        _phase_c_body,
        out_shape=jax.ShapeDtypeStruct((N, D), jnp.float32),
    )(num, den)


@jax.jit
def _run(x, edge_index, turn, tval, W_w, W_b, a_w, a_b):
    wh, p, q, mst = _phase_a(x, W_w, W_b, a_w, a_b)

    src = edge_index[0].reshape(NW, TPW)
    dst = edge_index[1].reshape(NW, TPW)
    trn = turn.reshape(NW, TPW)
    pad = TPAD - TPW
    srcp = jnp.pad(src, ((0, 0), (0, pad))).reshape(NW, NGRP, 8, BLK)
    dstp = jnp.pad(dst, ((0, 0), (0, pad)),
                   constant_values=N).reshape(NW, NGRP, 8, BLK)
    turnp = jnp.pad(trn, ((0, 0), (0, pad))).reshape(NW, NGRP, 8, BLK)
    tv = jnp.full((16,), tval, jnp.int32)

    num, den = _phase_b(wh, p, q, srcp, dstp, turnp, tv, mst)
    den = den.reshape(NW, NPAD)
    return _phase_c(num[:, :N, :], den[:, :N])


def kernel(x, edge_index, turn, t, offset, W_w, W_b, a_w, a_b):
    return _run(x, edge_index, turn, jnp.int32(t) + jnp.int32(offset),
                W_w, W_b, a_w, a_b)


# speedup vs baseline: 1.3792x; 1.3792x over previous
import functools
import jax
import jax.numpy as jnp
from jax import lax
from jax.experimental import pallas as pl
from jax.experimental.pallas import tpu as pltpu
from jax.experimental.pallas import tpu_sc as plsc

N = 10000
E = 320000
D = 128
ALPHA = 0.2

NC = 2
NS = 16
NW = NC * NS
TPW = E // NW
NBLK = 80
NGRP = NBLK // 8
TPAD = NBLK * 128
NPAD = 79 * 128


def _phase_a_body(x_ref, ww_ref, wb_ref, aw_ref, ab_ref, wh_ref, p_ref, q_ref,
                  mst_ref):
    x = x_ref[...]
    wh = lax.dot_general(x, ww_ref[...], (((1,), (1,)), ((), ())),
                         preferred_element_type=jnp.float32)
    wh = wh + wb_ref[...][None, :]
    wh_ref[...] = wh
    a2 = aw_ref[...].reshape(2, D)
    pq = lax.dot_general(wh, a2, (((1,), (1,)), ((), ())),
                         preferred_element_type=jnp.float32)
    p = pq[:, 0] + ab_ref[0]
    q = pq[:, 1]
    pad = jnp.zeros((NPAD - N,), jnp.float32)
    p_ref[...] = jnp.concatenate([p, pad])
    q_ref[...] = jnp.concatenate([q, pad])
    m = jnp.max(p) + jnp.max(q)
    m = jnp.where(m < 0, ALPHA * m, m)
    mst_ref[...] = jnp.full((16,), m, jnp.float32)


def _phase_a(x, W_w, W_b, a_w, a_b):
    return pl.pallas_call(
        _phase_a_body,
        out_shape=(
            jax.ShapeDtypeStruct((N, D), jnp.float32),
            jax.ShapeDtypeStruct((NPAD,), jnp.float32),
            jax.ShapeDtypeStruct((NPAD,), jnp.float32),
            jax.ShapeDtypeStruct((16,), jnp.float32),
        ),
    )(x, W_w, W_b, a_w, a_b)


def _sc_body(wh_hbm, p_hbm, q_hbm, src_hbm, dst_hbm, turn_hbm, tv_hbm,
             mst_hbm, num_out, den_out,
             num_sh, idx_s, idx_d, idx_t, pg0, qg0, pg1, qg1, den_v,
             rows0, rows1, mst_v, tv_v,
             semr0, semr1, semp0, semp1, semsc0, semsc1, semx):
    cid = lax.axis_index("c")
    sid = lax.axis_index("s")
    wid = cid * NS + sid

    pltpu.sync_copy(mst_hbm, mst_v)
    pltpu.sync_copy(tv_hbm, tv_v)

    zeros16f = jnp.zeros((16,), jnp.float32)

    def _zero_den(i, _):
        den_v[pl.ds(i * 16, 16)] = zeros16f
        return 0
    lax.fori_loop(0, NPAD // 16, _zero_den, 0)

    def _zero_rows(i, _):
        r = i // 8
        c = (i % 8) * 16
        rows0[r, pl.ds(c, 16)] = zeros16f
        return 0
    lax.fori_loop(0, 128 * 8, _zero_rows, 0)

    base = sid * (NPAD // NS)
    pltpu.sync_copy(rows0, num_sh.at[pl.ds(base, 128)])
    pltpu.sync_copy(rows0, num_sh.at[pl.ds(base + 128, 128)])
    pltpu.sync_copy(rows0, num_sh.at[pl.ds(base + 256, 128)])
    pltpu.sync_copy(rows0, num_sh.at[pl.ds(base + 384, 128)])
    pltpu.sync_copy(rows0.at[pl.ds(0, 120)], num_sh.at[pl.ds(base + 512, 120)])

    mst = mst_v[...]
    tv = tv_v[...]

    plsc.subcore_barrier()


    def issue_group_load(gi):
        gb = gi % 2
        pltpu.async_copy(src_hbm.at[wid, gi], idx_s.at[gb], semx)
        pltpu.async_copy(dst_hbm.at[wid, gi], idx_d.at[gb], semx)
        pltpu.async_copy(turn_hbm.at[wid, gi], idx_t.at[gb], semx)

    def wait_group_load():
        pltpu.make_async_copy(src_hbm.at[wid, 0], idx_s.at[0], semx).wait()
        pltpu.make_async_copy(dst_hbm.at[wid, 0], idx_d.at[0], semx).wait()
        pltpu.make_async_copy(turn_hbm.at[wid, 0], idx_t.at[0], semx).wait()

    def issue_gather(j, rows_ref, semr, semp, pgr, qgr):
        gp = (j // 8) % 2
        jj = j % 8
        pltpu.async_copy(p_hbm.at[idx_s.at[gp, jj]], pgr, semp)
        pltpu.async_copy(q_hbm.at[idx_d.at[gp, jj]], qgr, semp)
        for o in range(4):
            pltpu.async_copy(
                wh_hbm.at[idx_s.at[gp, jj, pl.ds(o * 32, 32)]],
                rows_ref.at[pl.ds(o * 32, 32)], semr)

    def wait_gather(rows_ref, semr, semp, pgr, qgr):
        pltpu.make_async_copy(p_hbm.at[idx_s.at[0, 0]], pgr, semp).wait()
        pltpu.make_async_copy(q_hbm.at[idx_d.at[0, 0]], qgr, semp).wait()
        for o in range(4):
            pltpu.make_async_copy(
                wh_hbm.at[idx_s.at[0, 0, pl.ds(0, 32)]],
                rows_ref.at[pl.ds(o * 32, 32)], semr).wait()

    def compute_scale(j, rows_ref, pgr, qgr):
        gp = (j // 8) % 2
        jj = j % 8

        def _sub(g, _c):
            c = g * 16
            pv = pgr[pl.ds(c, 16)]
            qv = qgr[pl.ds(c, 16)]
            d16 = idx_d[gp, jj, pl.ds(c, 16)]
            t16 = idx_t[gp, jj, pl.ds(c, 16)]
            e = pv + qv
            e = jnp.where(e < 0, ALPHA * e, e)
            w = jnp.exp(e - mst)
            w = jnp.where((t16 == tv) & (d16 < N), w, 0.0)
            plsc.addupdate_scatter(den_v, [d16], w)
            for i in range(16):
                wsc = w[i]
                r = c + i
                for k in range(8):
                    sl = pl.ds(k * 16, 16)
                    rows_ref[r, sl] = rows_ref[r, sl] * wsc
            return 0
        lax.fori_loop(0, 8, _sub, 0)

    def issue_scatter(j, rows_ref, semsc):
        gp = (j // 8) % 2
        jj = j % 8
        pltpu.async_copy(rows_ref, num_sh.at[idx_d.at[gp, jj]], semsc,
                         add=True)

    def drain_scatter(rows_ref, semsc):
        pltpu.make_async_copy(rows_ref, num_sh.at[pl.ds(0, 128)],
                              semsc).wait()

    pltpu.sync_copy(src_hbm.at[wid, 0], idx_s.at[0])
    pltpu.sync_copy(dst_hbm.at[wid, 0], idx_d.at[0])
    pltpu.sync_copy(turn_hbm.at[wid, 0], idx_t.at[0])
    issue_group_load(1)
    issue_gather(0, rows0, semr0, semp0, pg0, qg0)

    def _pair(it, _):
        j0 = 2 * it
        j1 = j0 + 1

        @pl.when(it > 0)
        def _():
            drain_scatter(rows1, semsc1)

        @pl.when((it % 4 == 0) & (it > 0) & (it // 4 + 1 <= NGRP - 1))
        def _():
            issue_group_load(it // 4 + 1)

        issue_gather(j1, rows1, semr1, semp1, pg1, qg1)
        wait_gather(rows0, semr0, semp0, pg0, qg0)
        compute_scale(j0, rows0, pg0, qg0)
        issue_scatter(j0, rows0, semsc0)

        @pl.when(j0 + 2 < NBLK)
        def _():
            @pl.when((j0 + 2) % 8 == 0)
            def _():
                wait_group_load()
            drain_scatter(rows0, semsc0)
            issue_gather(j0 + 2, rows0, semr0, semp0, pg0, qg0)

        wait_gather(rows1, semr1, semp1, pg1, qg1)
        compute_scale(j1, rows1, pg1, qg1)
        issue_scatter(j1, rows1, semsc1)
        return 0
    lax.fori_loop(0, NBLK // 2, _pair, 0)

    drain_scatter(rows0, semsc0)
    drain_scatter(rows1, semsc1)

    pltpu.sync_copy(den_v, den_out.at[pl.ds(wid * NPAD, NPAD)])

    plsc.subcore_barrier()

    rows_per_tile = NPAD // NS
    pltpu.sync_copy(num_sh.at[pl.ds(base, rows_per_tile)],
                    num_out.at[cid, pl.ds(base, rows_per_tile)])


def _phase_b(wh, p, q, srcp, dstp, turnp, tv, mst):
    mesh = plsc.VectorSubcoreMesh(core_axis_name="c", subcore_axis_name="s",
                                  num_cores=NC, num_subcores=NS)
    f = pl.kernel(
        _sc_body,
        out_type=(
            jax.ShapeDtypeStruct((NC, NPAD, D), jnp.float32),
            jax.ShapeDtypeStruct((NW * NPAD,), jnp.float32),
        ),
        mesh=mesh,
        scratch_types=[
            pltpu.VMEM_SHARED((NPAD, D), jnp.float32),
            pltpu.VMEM((2, 8, 128), jnp.int32),
            pltpu.VMEM((2, 8, 128), jnp.int32),
            pltpu.VMEM((2, 8, 128), jnp.int32),
            pltpu.VMEM((128,), jnp.float32),
            pltpu.VMEM((128,), jnp.float32),
            pltpu.VMEM((128,), jnp.float32),
            pltpu.VMEM((128,), jnp.float32),
            pltpu.VMEM((NPAD,), jnp.float32),
            pltpu.VMEM((128, D), jnp.float32),
            pltpu.VMEM((128, D), jnp.float32),
            pltpu.VMEM((16,), jnp.float32),
            pltpu.VMEM((16,), jnp.int32),
            pltpu.SemaphoreType.DMA,
            pltpu.SemaphoreType.DMA,
            pltpu.SemaphoreType.DMA,
            pltpu.SemaphoreType.DMA,
            pltpu.SemaphoreType.DMA,
            pltpu.SemaphoreType.DMA,
            pltpu.SemaphoreType.DMA,
        ],
        compiler_params=pltpu.CompilerParams(needs_layout_passes=False),
    )
    return f(wh, p, q, srcp, dstp, turnp, tv, mst)


def _phase_c_body(num_ref, den_ref, out_ref):
    num = num_ref[0] + num_ref[1]
    den = jnp.sum(den_ref[...], axis=0)
    out_ref[...] = num / (den[:, None] + 1e-9)


def _phase_c(num, den):
    return pl.pallas_call(
        _phase_c_body,
        out_shape=jax.ShapeDtypeStruct((N, D), jnp.float32),
    )(num, den)


@jax.jit
def _run(x, edge_index, turn, tval, W_w, W_b, a_w, a_b):
    wh, p, q, mst = _phase_a(x, W_w, W_b, a_w, a_b)

    src = edge_index[0].reshape(NW, TPW)
    dst = edge_index[1].reshape(NW, TPW)
    trn = turn.reshape(NW, TPW)
    pad = TPAD - TPW
    srcp = jnp.pad(src, ((0, 0), (0, pad))).reshape(NW, NGRP, 8, 128)
    dstp = jnp.pad(dst, ((0, 0), (0, pad)),
                   constant_values=N).reshape(NW, NGRP, 8, 128)
    turnp = jnp.pad(trn, ((0, 0), (0, pad))).reshape(NW, NGRP, 8, 128)
    tv = jnp.full((16,), tval, jnp.int32)

    num, den = _phase_b(wh, p, q, srcp, dstp, turnp, tv, mst)
    den = den.reshape(NW, NPAD)
    return _phase_c(num[:, :N, :], den[:, :N])


def kernel(x, edge_index, turn, t, offset, W_w, W_b, a_w, a_b):
    return _run(x, edge_index, turn, jnp.int32(t) + jnp.int32(offset),
                W_w, W_b, a_w, a_b)
